# baseline (speedup 1.0000x reference)
import functools

import jax
import jax.numpy as jnp
from jax import lax
from jax.experimental import pallas as pl
from jax.experimental.pallas import tpu as pltpu
from jax.experimental.pallas import tpu_sc as plsc

NC = 2
NS = 16
L = 16
NW = NC * NS
BATCH = 16384
DIM = 64
BPW = BATCH // NW
CHUNK = 128
NCHUNK = BPW // CHUNK
UNROLL = L

_mesh = plsc.VectorSubcoreMesh(core_axis_name="c", subcore_axis_name="s")


def _rsqrt(s):
    s = jnp.maximum(s, jnp.float32(1e-24))
    i = lax.bitcast_convert_type(s, jnp.int32)
    i = jnp.int32(0x5F3759DF) - (i >> 1)
    y = lax.bitcast_convert_type(i, jnp.float32)
    for _ in range(3):
        y = y * (jnp.float32(1.5) - jnp.float32(0.5) * s * y * y)
    return y


@functools.partial(
    pl.kernel,
    out_type=jax.ShapeDtypeStruct((BATCH,), jnp.float32),
    mesh=_mesh,
    compiler_params=pltpu.CompilerParams(needs_layout_passes=False,
                                         use_tc_tiling_on_sc=False),
    scratch_types=[
        pltpu.VMEM((NCHUNK, CHUNK), jnp.int32),
        pltpu.VMEM((NCHUNK, CHUNK), jnp.int32),
        pltpu.VMEM((NCHUNK, CHUNK), jnp.int32),
        pltpu.VMEM((BPW, DIM), jnp.float32),
        pltpu.VMEM((BPW, DIM), jnp.float32),
        pltpu.VMEM((BPW, DIM), jnp.float32),
        pltpu.VMEM((BPW,), jnp.float32),
        pltpu.SemaphoreType.DMA,
    ],
)
def _transe_kernel(hidx_hbm, ridx_hbm, tidx_hbm, ent_hbm, rel_hbm, out_hbm,
                   hidx_v, ridx_v, tidx_v, head_v, relrow_v, tail_v, out_v,
                   sem):
    wid = lax.axis_index("s") * NC + lax.axis_index("c")

    pltpu.sync_copy(hidx_hbm.at[pl.ds(wid * NCHUNK, NCHUNK)], hidx_v)
    pltpu.sync_copy(ridx_hbm.at[pl.ds(wid * NCHUNK, NCHUNK)], ridx_v)
    pltpu.sync_copy(tidx_hbm.at[pl.ds(wid * NCHUNK, NCHUNK)], tidx_v)

    copies = []
    for c in range(NCHUNK):
        sl = pl.ds(c * CHUNK, CHUNK)
        copies.append(pltpu.async_copy(ent_hbm.at[hidx_v.at[c]],
                                       head_v.at[sl], sem))
        copies.append(pltpu.async_copy(rel_hbm.at[ridx_v.at[c]],
                                       relrow_v.at[sl], sem))
        copies.append(pltpu.async_copy(ent_hbm.at[tidx_v.at[c]],
                                       tail_v.at[sl], sem))
    for cp in copies:
        cp.wait()

    lanes = lax.iota(jnp.int32, L)

    def body(it, carry):
        vec = jnp.zeros((L,), jnp.float32)
        for u in range(UNROLL):
            i = it * UNROLL + u
            h = [head_v[i, pl.ds(L * k, L)] for k in range(DIM // L)]
            r = [relrow_v[i, pl.ds(L * k, L)] for k in range(DIM // L)]
            t = [tail_v[i, pl.ds(L * k, L)] for k in range(DIM // L)]
            hs = h[0] * h[0] + h[1] * h[1] + h[2] * h[2] + h[3] * h[3]
            ts = t[0] * t[0] + t[1] * t[1] + t[2] * t[2] + t[3] * t[3]
            ih = _rsqrt(jnp.sum(hs))
            itn = _rsqrt(jnp.sum(ts))
            acc = None
            for k in range(DIM // L):
                term = jnp.abs(h[k] * ih + r[k] - t[k] * itn + 1e-6)
                acc = term if acc is None else acc + term
            vec = jnp.where(lanes == u, jnp.sum(acc), vec)
        out_v[pl.ds(it * UNROLL, UNROLL)] = vec
        return carry

    lax.fori_loop(0, BPW // UNROLL, body, 0)

    pltpu.sync_copy(out_v, out_hbm.at[pl.ds(wid * BPW, BPW)])


def kernel(triplet_idx, entity_emb, relation_emb):
    hidx = triplet_idx[:, 0].reshape(NW * NCHUNK, CHUNK)
    ridx = triplet_idx[:, 1].reshape(NW * NCHUNK, CHUNK)
    tidx = triplet_idx[:, 2].reshape(NW * NCHUNK, CHUNK)
    return _transe_kernel(hidx, ridx, tidx, entity_emb, relation_emb)
